# Initial kernel scaffold; baseline (speedup 1.0000x reference)
import jax
import jax.numpy as jnp
from jax.experimental import pallas as pl


def kernel(x_cat, x_con, emb, W1, b1, W2, b2, Wout, bout):
    raise NotImplementedError("write your pallas kernel here")


# speedup vs baseline: 2.0756x; 2.0756x over previous
import functools

import jax
import jax.numpy as jnp
from jax import lax
from jax.experimental import pallas as pl
from jax.experimental.pallas import tpu as pltpu
from jax.experimental.pallas import tpu_sc as plsc

_B = 16384
_F = 26
_VOCAB = 100000
_ED = 32
_NC_CON = 13
_HD = 64
_NOUT = 2

_SC_CORES = 2
_SC_SUBCORES = 16
_NW = _SC_CORES * _SC_SUBCORES
_ROWS_PER_W = _B * _F // _NW
_CHUNK = 1024
_NCHUNK = _ROWS_PER_W // _CHUNK


def _sc_gather_body(table_hbm, idx_hbm, out_hbm, idx_v, buf0, buf1,
                    gsem0, gsem1):
    wid = lax.axis_index("s") * _SC_CORES + lax.axis_index("c")
    base = wid * _ROWS_PER_W
    pltpu.sync_copy(idx_hbm.at[pl.ds(base, _ROWS_PER_W)], idx_v)

    bufs = (buf0, buf1)
    gsems = (gsem0, gsem1)
    handles = [None, None]
    handles[0] = pltpu.async_copy(
        table_hbm.at[idx_v.at[pl.ds(0, _CHUNK)]], bufs[0], gsems[0])
    for c in range(_NCHUNK):
        cur = c % 2
        nxt = (c + 1) % 2
        if c + 1 < _NCHUNK:
            handles[nxt] = pltpu.async_copy(
                table_hbm.at[idx_v.at[pl.ds((c + 1) * _CHUNK, _CHUNK)]],
                bufs[nxt], gsems[nxt])
        handles[cur].wait()
        pltpu.sync_copy(bufs[cur],
                        out_hbm.at[pl.ds(base + c * _CHUNK, _CHUNK)])


@jax.jit
def _sc_gather(table, idx):
    mesh = plsc.VectorSubcoreMesh(
        core_axis_name="c", subcore_axis_name="s",
        num_cores=_SC_CORES, num_subcores=_SC_SUBCORES)
    return pl.kernel(
        _sc_gather_body,
        out_type=jax.ShapeDtypeStruct((_B * _F, _ED), jnp.float32),
        mesh=mesh,
        scratch_types=[
            pltpu.VMEM((_ROWS_PER_W,), jnp.int32),
            pltpu.VMEM((_CHUNK, _ED), jnp.float32),
            pltpu.VMEM((_CHUNK, _ED), jnp.float32),
            pltpu.SemaphoreType.DMA,
            pltpu.SemaphoreType.DMA,
        ],
        compiler_params=pltpu.CompilerParams(use_tc_tiling_on_sc=False),
    )(table, idx)


def _mlp_body(xe_ref, xc_ref, w1_ref, b1_ref, w2_ref, b2_ref, wout_ref,
              bout_ref, o_ref):
    h1 = jnp.dot(xe_ref[...], w1_ref[...], preferred_element_type=jnp.float32)
    h1 = jnp.maximum(h1 + b1_ref[...], 0.0)
    h2 = jnp.dot(xc_ref[...], w2_ref[...], preferred_element_type=jnp.float32)
    h2 = jnp.maximum(h2 + b2_ref[...], 0.0)
    o = jnp.dot(h2, wout_ref[0:_HD, :], preferred_element_type=jnp.float32)
    o += jnp.dot(h1, wout_ref[_HD:2 * _HD, :],
                 preferred_element_type=jnp.float32)
    o_ref[...] = o + bout_ref[...]


_BLK = 2048


@jax.jit
def _mlp(xe, xc, W1, b1, W2, b2, Wout, bout):
    grid = (_B // _BLK,)
    return pl.pallas_call(
        _mlp_body,
        grid=grid,
        in_specs=[
            pl.BlockSpec((_BLK, _F * _ED), lambda i: (i, 0)),
            pl.BlockSpec((_BLK, _NC_CON), lambda i: (i, 0)),
            pl.BlockSpec((_F * _ED, _HD), lambda i: (0, 0)),
            pl.BlockSpec((1, _HD), lambda i: (0, 0)),
            pl.BlockSpec((_NC_CON, _HD), lambda i: (0, 0)),
            pl.BlockSpec((1, _HD), lambda i: (0, 0)),
            pl.BlockSpec((2 * _HD, _NOUT), lambda i: (0, 0)),
            pl.BlockSpec((1, _NOUT), lambda i: (0, 0)),
        ],
        out_specs=pl.BlockSpec((_BLK, _NOUT), lambda i: (i, 0)),
        out_shape=jax.ShapeDtypeStruct((_B, _NOUT), jnp.float32),
    )(xe, xc, W1, b1, W2, b2, Wout, bout)


def kernel(x_cat, x_con, emb, W1, b1, W2, b2, Wout, bout):
    table = emb.reshape(_F * _VOCAB, _ED)
    offs = (jnp.arange(_F, dtype=jnp.int32) * _VOCAB)[None, :]
    idx = (x_cat + offs).reshape(-1)
    xe = _sc_gather(table, idx)
    xe = xe.reshape(_B, _F * _ED)
    return _mlp(xe, x_con, W1, b1.reshape(1, -1), W2, b2.reshape(1, -1),
                Wout, bout.reshape(1, -1))


# speedup vs baseline: 2.0790x; 1.0017x over previous
import functools

import jax
import jax.numpy as jnp
from jax import lax
from jax.experimental import pallas as pl
from jax.experimental.pallas import tpu as pltpu
from jax.experimental.pallas import tpu_sc as plsc

_B = 16384
_F = 26
_VOCAB = 100000
_ED = 32
_NC_CON = 13
_HD = 64
_NOUT = 2

_SC_CORES = 2
_SC_SUBCORES = 16
_NW = _SC_CORES * _SC_SUBCORES
_ROWS_PER_W = _B * _F // _NW
_CHUNK = 1024
_NCHUNK = _ROWS_PER_W // _CHUNK


_NBUF = 3


def _sc_gather_body(table_hbm, idx_hbm, out_hbm, idx_v, *rest):
    bufs = rest[:_NBUF]
    gsems = rest[_NBUF:2 * _NBUF]
    osems = rest[2 * _NBUF:3 * _NBUF]
    wid = lax.axis_index("s") * _SC_CORES + lax.axis_index("c")
    base = wid * _ROWS_PER_W
    pltpu.sync_copy(idx_hbm.at[pl.ds(base, _ROWS_PER_W)], idx_v)

    go = [None] * _NBUF
    oo = [None] * _NBUF

    def fire_gather(c, b):
        return pltpu.async_copy(
            table_hbm.at[idx_v.at[pl.ds(c * _CHUNK, _CHUNK)]],
            bufs[b], gsems[b])

    def fire_out(c, b):
        return pltpu.async_copy(
            bufs[b], out_hbm.at[pl.ds(base + c * _CHUNK, _CHUNK)], osems[b])

    lag = _NBUF - 1
    for c in range(_NCHUNK):
        b = c % _NBUF
        if oo[b] is not None:
            oo[b].wait()
        go[b] = fire_gather(c, b)
        d = c - lag
        if d >= 0:
            bb = d % _NBUF
            go[bb].wait()
            oo[bb] = fire_out(d, bb)
    for d in range(max(0, _NCHUNK - lag), _NCHUNK):
        bb = d % _NBUF
        go[bb].wait()
        oo[bb] = fire_out(d, bb)
    for b in range(_NBUF):
        if oo[b] is not None:
            oo[b].wait()


@jax.jit
def _sc_gather(table, idx):
    mesh = plsc.VectorSubcoreMesh(
        core_axis_name="c", subcore_axis_name="s",
        num_cores=_SC_CORES, num_subcores=_SC_SUBCORES)
    return pl.kernel(
        _sc_gather_body,
        out_type=jax.ShapeDtypeStruct((_B * _F, _ED), jnp.float32),
        mesh=mesh,
        scratch_types=(
            [pltpu.VMEM((_ROWS_PER_W,), jnp.int32)]
            + [pltpu.VMEM((_CHUNK, _ED), jnp.float32)] * _NBUF
            + [pltpu.SemaphoreType.DMA] * (2 * _NBUF)
        ),
        compiler_params=pltpu.CompilerParams(use_tc_tiling_on_sc=False),
    )(table, idx)


def _mlp_body(xe_ref, xc_ref, w1_ref, b1_ref, w2_ref, b2_ref, wout_ref,
              bout_ref, o_ref):
    h1 = jnp.dot(xe_ref[...], w1_ref[...], preferred_element_type=jnp.float32)
    h1 = jnp.maximum(h1 + b1_ref[...], 0.0)
    h2 = jnp.dot(xc_ref[...], w2_ref[...], preferred_element_type=jnp.float32)
    h2 = jnp.maximum(h2 + b2_ref[...], 0.0)
    o = jnp.dot(h2, wout_ref[0:_HD, :], preferred_element_type=jnp.float32)
    o += jnp.dot(h1, wout_ref[_HD:2 * _HD, :],
                 preferred_element_type=jnp.float32)
    o_ref[...] = o + bout_ref[...]


_BLK = 2048


@jax.jit
def _mlp(xe, xc, W1, b1, W2, b2, Wout, bout):
    grid = (_B // _BLK,)
    return pl.pallas_call(
        _mlp_body,
        grid=grid,
        in_specs=[
            pl.BlockSpec((_BLK, _F * _ED), lambda i: (i, 0)),
            pl.BlockSpec((_BLK, _NC_CON), lambda i: (i, 0)),
            pl.BlockSpec((_F * _ED, _HD), lambda i: (0, 0)),
            pl.BlockSpec((1, _HD), lambda i: (0, 0)),
            pl.BlockSpec((_NC_CON, _HD), lambda i: (0, 0)),
            pl.BlockSpec((1, _HD), lambda i: (0, 0)),
            pl.BlockSpec((2 * _HD, _NOUT), lambda i: (0, 0)),
            pl.BlockSpec((1, _NOUT), lambda i: (0, 0)),
        ],
        out_specs=pl.BlockSpec((_BLK, _NOUT), lambda i: (i, 0)),
        out_shape=jax.ShapeDtypeStruct((_B, _NOUT), jnp.float32),
    )(xe, xc, W1, b1, W2, b2, Wout, bout)


def kernel(x_cat, x_con, emb, W1, b1, W2, b2, Wout, bout):
    table = emb.reshape(_F * _VOCAB, _ED)
    offs = (jnp.arange(_F, dtype=jnp.int32) * _VOCAB)[None, :]
    idx = (x_cat + offs).reshape(-1)
    xe = _sc_gather(table, idx)
    xe = xe.reshape(_B, _F * _ED)
    return _mlp(xe, x_con, W1, b1.reshape(1, -1), W2, b2.reshape(1, -1),
                Wout, bout.reshape(1, -1))
